# baseline (speedup 1.0000x reference)
import jax
import jax.numpy as jnp
from jax.experimental import pallas as pl
from jax.experimental.pallas import tpu as pltpu

_BN = 4096
_K = 8
_BQ = _BN // _K


def _tc_block(t_ref, *refs):
    x_refs = refs[:_K]
    lbl_ref, w_ref, out_ref = refs[_K], refs[_K + 1], refs[_K + 2]
    C = x_refs[0].shape[0]
    inv_t = 1.0 / t_ref[0]
    part = jnp.float32(0.0)
    for k in range(_K):
        x = x_refs[k][...]
        e = jnp.exp(x * inv_t)
        s1 = jnp.sum(e, axis=0, keepdims=True)
        s2 = jnp.sum(e * e, axis=0, keepdims=True)
        rows = jax.lax.broadcasted_iota(jnp.int32, x.shape, 0)
        lbl = lbl_ref[:, k * _BQ:(k + 1) * _BQ]
        el = jnp.sum(jnp.where(rows == lbl, e, 0.0), axis=0, keepdims=True)
        brier = (s2 + (s1 - 2.0 * el) * s1) / (s1 * s1)
        part = part + jnp.sum(brier * w_ref[:, k * _BQ:(k + 1) * _BQ])
    prev = jnp.where(pl.program_id(0) == 0, 0.0, out_ref[0, 0])
    out_ref[...] = jnp.full((8, 128), prev + part, jnp.float32)


def kernel(logits, labels, weight, T):
    B, C = logits.shape
    xt = logits.T
    lbl = labels.astype(jnp.int32).reshape(1, B)
    wt = weight.reshape(1, B)
    grid = B // _BN
    x_specs = [
        pl.BlockSpec((C, _BQ), lambda i, k=k: (0, i * _K + k)) for k in range(_K)
    ]
    acc = pl.pallas_call(
        _tc_block,
        grid=(grid,),
        in_specs=[pl.BlockSpec(memory_space=pltpu.SMEM)]
        + x_specs
        + [
            pl.BlockSpec((1, _BN), lambda i: (0, i)),
            pl.BlockSpec((1, _BN), lambda i: (0, i)),
        ],
        out_specs=pl.BlockSpec((8, 128), lambda i: (0, 0)),
        out_shape=jax.ShapeDtypeStruct((8, 128), jnp.float32),
    )(T, *([xt] * _K), lbl, wt)
    return acc[0, 0] / B


# speedup vs baseline: 1.0457x; 1.0457x over previous
import jax
import jax.numpy as jnp
from jax.experimental import pallas as pl
from jax.experimental.pallas import tpu as pltpu

_BN = 2048
_K = 16
_BQ = _BN // _K


def _tc_block(t_ref, *refs):
    x_refs = refs[:_K]
    lbl_ref, w_ref, out_ref = refs[_K], refs[_K + 1], refs[_K + 2]
    C = x_refs[0].shape[0]
    inv_t = 1.0 / t_ref[0]
    part = jnp.float32(0.0)
    for k in range(_K):
        x = x_refs[k][...]
        e = jnp.exp(x * inv_t)
        s1 = jnp.sum(e, axis=0, keepdims=True)
        s2 = jnp.sum(e * e, axis=0, keepdims=True)
        rows = jax.lax.broadcasted_iota(jnp.int32, x.shape, 0)
        lbl = lbl_ref[:, k * _BQ:(k + 1) * _BQ]
        el = jnp.sum(jnp.where(rows == lbl, e, 0.0), axis=0, keepdims=True)
        brier = (s2 + (s1 - 2.0 * el) * s1) / (s1 * s1)
        part = part + jnp.sum(brier * w_ref[:, k * _BQ:(k + 1) * _BQ])
    prev = jnp.where(pl.program_id(0) == 0, 0.0, out_ref[0, 0])
    out_ref[...] = jnp.full((8, 128), prev + part, jnp.float32)


def kernel(logits, labels, weight, T):
    B, C = logits.shape
    xt = logits.T
    lbl = labels.astype(jnp.int32).reshape(1, B)
    wt = weight.reshape(1, B)
    grid = B // _BN
    x_specs = [
        pl.BlockSpec((C, _BQ), lambda i, k=k: (0, i * _K + k)) for k in range(_K)
    ]
    acc = pl.pallas_call(
        _tc_block,
        grid=(grid,),
        in_specs=[pl.BlockSpec(memory_space=pltpu.SMEM)]
        + x_specs
        + [
            pl.BlockSpec((1, _BN), lambda i: (0, i)),
            pl.BlockSpec((1, _BN), lambda i: (0, i)),
        ],
        out_specs=pl.BlockSpec((8, 128), lambda i: (0, 0)),
        out_shape=jax.ShapeDtypeStruct((8, 128), jnp.float32),
    )(T, *([xt] * _K), lbl, wt)
    return acc[0, 0] / B
